# baseline (speedup 1.0000x reference)
import jax
import jax.numpy as jnp
from jax import lax
from jax.experimental import pallas as pl
from jax.experimental.pallas import tpu as pltpu
from jax.experimental.pallas import tpu_sc as plsc

N = 10000
E = 320000
FEATS = 128
HIDDEN = 64
C = 64
NTRAIN = 1000
DEPTH = 10

NSUB = 16
RPT = 640
NPAD = NSUB * RPT
QCH = 128
NQ = RPT // QCH
ECH = 128
NCHUNK = 162
NCH = NCHUNK // 2
NSLOT = 3
EPAD = NSUB * NCHUNK * ECH
TPT = 64
PAD_ROW = N + 64

_Z16 = lambda: jnp.zeros((16,), jnp.float32)


def _mlp_body(x_ref, w1_ref, b1_ref, w2_ref, b2_ref, p_ref):
    h = jnp.maximum(
        jnp.dot(x_ref[...], w1_ref[...], preferred_element_type=jnp.float32)
        + b1_ref[...], 0.0)
    logits = (jnp.dot(h, w2_ref[...], preferred_element_type=jnp.float32)
              + b2_ref[...])
    m = jnp.max(logits, axis=1, keepdims=True)
    e = jnp.exp(logits - m)
    p_ref[...] = e / jnp.sum(e, axis=1, keepdims=True)


def _mlp_softmax(x, W1, b1, W2, b2):
    blk = 400
    return pl.pallas_call(
        _mlp_body,
        grid=(N // blk,),
        in_specs=[
            pl.BlockSpec((blk, FEATS), lambda i: (i, 0)),
            pl.BlockSpec((FEATS, HIDDEN), lambda i: (0, 0)),
            pl.BlockSpec((1, HIDDEN), lambda i: (0, 0)),
            pl.BlockSpec((HIDDEN, C), lambda i: (0, 0)),
            pl.BlockSpec((1, C), lambda i: (0, 0)),
        ],
        out_specs=pl.BlockSpec((blk, C), lambda i: (i, 0)),
        out_shape=jax.ShapeDtypeStruct((N, C), jnp.float32),
    )(x, W1, b1.reshape(1, HIDDEN), W2, b2.reshape(1, C))


def _log1p_body(x_ref, o_ref):
    o_ref[...] = jnp.log(x_ref[...] + 1.0)


def _log1p(x):
    blk = 400
    return pl.pallas_call(
        _log1p_body,
        grid=(N // blk,),
        in_specs=[pl.BlockSpec((blk, C), lambda i: (i, 0))],
        out_specs=pl.BlockSpec((blk, C), lambda i: (i, 0)),
        out_shape=jax.ShapeDtypeStruct((N, C), jnp.float32),
    )(x)


def _sc_body(p_hbm, sdp, tlp, cur2_hbm, h0b2_hbm, dinv_hbm, px_hbm, flg_hbm,
             acc,
             gbuf, wbuf, hbuf, h0t, sdbuf, tl, semg, sems, xsem):
    core = lax.axis_index("c")
    sid = lax.axis_index("s")
    base = sid * RPT
    cur_h = cur2_hbm.at[core]
    h0b_hbm = h0b2_hbm.at[core]

    def xbarrier(t):
        plsc.subcore_barrier()

        @pl.when(sid == 0)
        def _():
            pltpu.semaphore_signal(xsem, 1, core_index=1 - core)
            pl.semaphore_wait(xsem, 1)
        plsc.subcore_barrier()

    def export_partial(slot):
        for q in range(NQ):
            to_wbuf(acc, q)
            pltpu.sync_copy(
                wbuf, px_hbm.at[core, slot].at[pl.ds(base + q * QCH, QCH)])

    def zero_wbuf():
        def zr(r, x):
            for cc in range(4):
                wbuf[r, pl.ds(cc * 16, 16)] = _Z16()
            return x
        lax.fori_loop(0, QCH, zr, None)

    def wbuf_to(ref, q):
        pltpu.sync_copy(wbuf, ref.at[pl.ds(base + q * QCH, QCH)])

    def to_wbuf(ref, q):
        pltpu.sync_copy(ref.at[pl.ds(base + q * QCH, QCH)], wbuf)

    gb0 = gbuf.at[0]
    gb1 = gbuf.at[1]

    def gb_lo():
        return gb0.at[pl.ds(0, QCH)]

    def gb_hi():
        return gb1.at[pl.ds(0, QCH)]

    pltpu.sync_copy(tlp.at[sid], tl)
    pltpu.sync_copy(sdp.at[sid].at[pl.ds(core * NCH, NCH)], sdbuf)

    zero_wbuf()
    for q in range(NQ):
        wbuf_to(acc, q)
    plsc.subcore_barrier()

    def fill_ones(r, x):
        for cc in range(4):
            gb0[r, pl.ds(cc * 16, 16)] = jnp.ones((16,), jnp.float32)
        return x
    lax.fori_loop(0, ECH, fill_ones, None)

    def deg_step(k, x):
        pltpu.sync_copy(gb0, acc.at[sdbuf.at[k].at[1]], add=True)
        return x
    lax.fori_loop(0, NCH, deg_step, None)
    plsc.subcore_barrier()
    export_partial(1)
    xbarrier(jnp.int32(1))

    for q in range(NQ):
        to_wbuf(acc, q)
        pltpu.sync_copy(
            px_hbm.at[1 - core, 1].at[pl.ds(base + q * QCH, QCH)], gb_hi())

        def dinv_row(r, x):
            for cc in range(4):
                sl = pl.ds(cc * 16, 16)
                wbuf[r, sl] = 1.0 / jnp.maximum(wbuf[r, sl] + gb1[r, sl],
                                                1.0)
            return x
        lax.fori_loop(0, QCH, dinv_row, None)
        pltpu.sync_copy(wbuf, dinv_hbm.at[pl.ds(base + q * QCH, QCH)])
        zero_wbuf()
        wbuf_to(acc, q)

    for q in range(NQ):
        pltpu.sync_copy(p_hbm.at[pl.ds(base + q * QCH, QCH)], gb_lo())

        def neg_row(r, x):
            for cc in range(4):
                sl = pl.ds(cc * 16, 16)
                wbuf[r, sl] = -gb0[r, sl]
            return x
        lax.fori_loop(0, QCH, neg_row, None)
        wbuf_to(cur_h, q)
    plsc.subcore_barrier()

    pltpu.sync_copy(cur_h.at[tl.at[0]], h0t)

    def h0t_row(k, x):
        lchunk = tl[1, pl.ds((k // 16) * 16, 16)]
        lval = lax.gather(
            lchunk, jnp.full((16, 1), k % 16, jnp.int32),
            dimension_numbers=lax.GatherDimensionNumbers(
                offset_dims=(), collapsed_slice_dims=(0,),
                start_index_map=(0,)),
            slice_sizes=(1,),
            mode=lax.GatherScatterMode.PROMISE_IN_BOUNDS)
        for cc in range(4):
            io = lax.iota(jnp.int32, 16) + (cc * 16)
            oh = jnp.where(io == lval, 1.0, 0.0)
            h0t[k, pl.ds(cc * 16, 16)] = oh + h0t[k, pl.ds(cc * 16, 16)]
        return x
    lax.fori_loop(0, TPT, h0t_row, None)
    plsc.subcore_barrier()

    pltpu.sync_copy(h0t, cur_h.at[tl.at[0]])
    plsc.subcore_barrier()

    def idx_ref(k, row):
        return sdbuf.at[k].at[row]

    def start_gather(k, s):
        pltpu.async_copy(cur_h.at[idx_ref(k, 0)], gbuf.at[s], semg.at[s])

    def wait_gather(k, s):
        pltpu.make_async_copy(cur_h.at[idx_ref(k, 0)], gbuf.at[s],
                              semg.at[s]).wait()

    def start_scat(k, s):
        pltpu.async_copy(gbuf.at[s], acc.at[idx_ref(k, 1)],
                         sems.at[s], add=True)

    def wait_scat(k, s):
        pltpu.make_async_copy(gbuf.at[s], acc.at[idx_ref(k, 1)],
                              sems.at[s]).wait()

    def scatter_phase():
        start_gather(0, 0)
        start_gather(1, 1)
        start_gather(2, 2)
        wait_gather(0, 0)
        start_scat(0, 0)

        def pipe(j3, x):
            j = 3 * j3
            wait_gather(j + 1, 1)
            start_scat(j + 1, 1)
            wait_scat(j, 0)
            start_gather(j + 3, 0)
            wait_gather(j + 2, 2)
            start_scat(j + 2, 2)
            wait_scat(j + 1, 1)
            start_gather(j + 4, 1)
            wait_gather(j + 3, 0)
            start_scat(j + 3, 0)
            wait_scat(j + 2, 2)
            start_gather(j + 5, 2)
            return x
        lax.fori_loop(0, (NCH - 6) // 3, pipe, None)
        j = NCH - 6
        wait_gather(j + 1, 1)
        start_scat(j + 1, 1)
        wait_scat(j, 0)
        wait_gather(j + 2, 2)
        start_scat(j + 2, 2)
        wait_scat(j + 1, 1)
        wait_gather(j + 3, 0)
        start_scat(j + 3, 0)
        wait_scat(j + 2, 2)
        wait_gather(j + 4, 1)
        start_scat(j + 4, 1)
        wait_scat(j + 3, 0)
        wait_gather(j + 5, 2)
        start_scat(j + 5, 2)
        wait_scat(j + 4, 1)
        wait_scat(j + 5, 2)

    def scale_chunk(q, slot, second):
        to_wbuf(acc, q)
        pltpu.sync_copy(dinv_hbm.at[pl.ds(base + q * QCH, QCH)], gb_lo())
        pltpu.sync_copy(
            px_hbm.at[1 - core, slot].at[pl.ds(base + q * QCH, QCH)],
            gb_hi())
        if second:
            pltpu.sync_copy(h0b_hbm.at[pl.ds(base + q * QCH, QCH)], hbuf)

        def srow(r, x):
            for cc in range(4):
                sl = pl.ds(cc * 16, 16)
                tot = (wbuf[r, sl] + gb1[r, sl]) * gb0[r, sl]
                if second:
                    wbuf[r, sl] = tot * 0.9 + hbuf[r, sl] * 0.1
                else:
                    wbuf[r, sl] = tot
            return x
        lax.fori_loop(0, QCH, srow, None)
        wbuf_to(cur_h, q)
        zero_wbuf()
        wbuf_to(acc, q)

    def loop1_step(s, carry):
        t = 2 + s
        slot = lax.rem(t, 2)
        scatter_phase()
        plsc.subcore_barrier()
        export_partial(slot)
        xbarrier(t)
        for q in range(NQ):
            scale_chunk(q, slot, False)
        plsc.subcore_barrier()
        pltpu.sync_copy(h0t, cur_h.at[tl.at[0]])
        plsc.subcore_barrier()
        return carry
    lax.fori_loop(0, DEPTH, loop1_step, None)

    for q in range(NQ):
        to_wbuf(cur_h, q)
        pltpu.sync_copy(p_hbm.at[pl.ds(base + q * QCH, QCH)], gb_lo())

        def arow(r, x):
            for cc in range(4):
                sl = pl.ds(cc * 16, 16)
                wbuf[r, sl] = wbuf[r, sl] + gb0[r, sl]
            return x
        lax.fori_loop(0, QCH, arow, None)
        wbuf_to(cur_h, q)
        pltpu.sync_copy(wbuf, h0b_hbm.at[pl.ds(base + q * QCH, QCH)])
    plsc.subcore_barrier()

    def loop2_step(s, carry):
        t = 12 + s
        slot = lax.rem(t, 2)
        scatter_phase()
        plsc.subcore_barrier()
        export_partial(slot)
        xbarrier(t)
        for q in range(NQ):
            scale_chunk(q, slot, True)
        plsc.subcore_barrier()
        return carry
    lax.fori_loop(0, DEPTH, loop2_step, None)


def _sc_diffusion(p_pad, sdp, tlp):
    mesh = plsc.VectorSubcoreMesh(core_axis_name="c", subcore_axis_name="s")
    kfn = pl.kernel(
        _sc_body,
        out_type=(jax.ShapeDtypeStruct((2, NPAD, C), jnp.float32),
                  jax.ShapeDtypeStruct((2, NPAD, C), jnp.float32),
                  jax.ShapeDtypeStruct((NPAD, C), jnp.float32),
                  jax.ShapeDtypeStruct((2, 2, NPAD, C), jnp.float32),
                  jax.ShapeDtypeStruct((2, 1, 64), jnp.int32)),
        mesh=mesh,
        compiler_params=pltpu.CompilerParams(use_tc_tiling_on_sc=False),
        scratch_types=[
            pltpu.VMEM_SHARED((NPAD, C), jnp.float32),
            pltpu.VMEM((NSLOT, ECH, C), jnp.float32),
            pltpu.VMEM((QCH, C), jnp.float32),
            pltpu.VMEM((QCH, C), jnp.float32),
            pltpu.VMEM((TPT, C), jnp.float32),
            pltpu.VMEM((NCH, 2, ECH), jnp.int32),
            pltpu.VMEM((2, TPT), jnp.int32),
            pltpu.SemaphoreType.DMA((NSLOT,)),
            pltpu.SemaphoreType.DMA((NSLOT,)),
            pltpu.SemaphoreType.REGULAR,
        ],
    )
    return kfn(p_pad, sdp, tlp)


def kernel(x, edge_index, train_idx, labels, W1, b1, W2, b2):
    p = _mlp_softmax(x, W1, b1, W2, b2)
    p_pad = jnp.pad(p, ((0, NPAD - N), (0, 0)))

    src = jnp.pad(edge_index[0].astype(jnp.int32), (0, EPAD - E))
    dst = jnp.pad(edge_index[1].astype(jnp.int32), (0, EPAD - E),
                  constant_values=PAD_ROW)
    sdp = jnp.stack([src.reshape(NSUB, NCHUNK, ECH),
                     dst.reshape(NSUB, NCHUNK, ECH)], axis=2)
    tlp = jnp.stack(
        [jnp.pad(train_idx.astype(jnp.int32), (0, NSUB * TPT - NTRAIN),
                 constant_values=PAD_ROW).reshape(NSUB, TPT),
         jnp.pad(labels.astype(jnp.int32),
                 (0, NSUB * TPT - NTRAIN)).reshape(NSUB, TPT)], axis=1)

    cur2, _, _, _, _ = _sc_diffusion(p_pad, sdp, tlp)
    out_pre = jnp.concatenate([cur2[0, :NPAD // 2], cur2[1, NPAD // 2:N]])
    return _log1p(out_pre)


# speedup vs baseline: 1.5043x; 1.5043x over previous
import jax
import jax.numpy as jnp
from jax import lax
from jax.experimental import pallas as pl
from jax.experimental.pallas import tpu as pltpu
from jax.experimental.pallas import tpu_sc as plsc

N = 10000
E = 320000
FEATS = 128
HIDDEN = 64
C = 64
NTRAIN = 1000
DEPTH = 10

NSUB = 16
RPT = 640
NPAD = NSUB * RPT
QCH = 128
NQ = RPT // QCH
ECH = 128
NCHUNK = 160
BF = 16
NB = NCHUNK // BF
NSLOT = 3
EPAD = NSUB * NCHUNK * ECH
TPT = 64
PAD_ROW = N + 64

_Z16 = lambda: jnp.zeros((16,), jnp.float32)


def _mlp_body(x_ref, w1_ref, b1_ref, w2_ref, b2_ref, p_ref):
    h = jnp.maximum(
        jnp.dot(x_ref[...], w1_ref[...], preferred_element_type=jnp.float32)
        + b1_ref[...], 0.0)
    logits = (jnp.dot(h, w2_ref[...], preferred_element_type=jnp.float32)
              + b2_ref[...])
    m = jnp.max(logits, axis=1, keepdims=True)
    e = jnp.exp(logits - m)
    p_ref[...] = e / jnp.sum(e, axis=1, keepdims=True)


def _mlp_softmax(x, W1, b1, W2, b2):
    blk = 400
    return pl.pallas_call(
        _mlp_body,
        grid=(N // blk,),
        in_specs=[
            pl.BlockSpec((blk, FEATS), lambda i: (i, 0)),
            pl.BlockSpec((FEATS, HIDDEN), lambda i: (0, 0)),
            pl.BlockSpec((1, HIDDEN), lambda i: (0, 0)),
            pl.BlockSpec((HIDDEN, C), lambda i: (0, 0)),
            pl.BlockSpec((1, C), lambda i: (0, 0)),
        ],
        out_specs=pl.BlockSpec((blk, C), lambda i: (i, 0)),
        out_shape=jax.ShapeDtypeStruct((N, C), jnp.float32),
    )(x, W1, b1.reshape(1, HIDDEN), W2, b2.reshape(1, C))


def _log1p_body(x_ref, o_ref):
    o_ref[...] = jnp.log(x_ref[...] + 1.0)


def _log1p(x):
    blk = 400
    return pl.pallas_call(
        _log1p_body,
        grid=(N // blk,),
        in_specs=[pl.BlockSpec((blk, C), lambda i: (i, 0))],
        out_specs=pl.BlockSpec((blk, C), lambda i: (i, 0)),
        out_shape=jax.ShapeDtypeStruct((N, C), jnp.float32),
    )(x)


def _sc_body(p_hbm, sdp, tlp, cur2_hbm, h0b2_hbm, dinv_hbm, px_hbm, flg_hbm,
             acc,
             gbuf, wbuf, h0t, sdbuf, tl, fbuf, fbuf2, semg, sems, semi,
             xsem):
    core = lax.axis_index("c")
    sid = lax.axis_index("s")
    base = sid * RPT
    cur_h = cur2_hbm.at[core]
    h0b_hbm = h0b2_hbm.at[core]
    NBH = NB // 2

    def xbarrier(t):
        plsc.subcore_barrier()

        @pl.when(sid == 0)
        def _():
            pltpu.semaphore_signal(xsem, 1, core_index=1 - core)
            pl.semaphore_wait(xsem, 1)
        plsc.subcore_barrier()

    def export_partial(slot):
        for q in range(NQ):
            to_wbuf(acc, q)
            pltpu.sync_copy(
                wbuf, px_hbm.at[core, slot].at[pl.ds(base + q * QCH, QCH)])

    def zero_wbuf():
        def zr(r, x):
            for cc in range(4):
                wbuf[r, pl.ds(cc * 16, 16)] = _Z16()
            return x
        lax.fori_loop(0, QCH, zr, None)

    def wbuf_to(ref, q):
        pltpu.sync_copy(wbuf, ref.at[pl.ds(base + q * QCH, QCH)])

    def to_wbuf(ref, q):
        pltpu.sync_copy(ref.at[pl.ds(base + q * QCH, QCH)], wbuf)

    gb0 = gbuf.at[0]
    gb1 = gbuf.at[1]

    def gb_lo():
        return gb0.at[pl.ds(0, QCH)]

    def gb_hi():
        return gb1.at[pl.ds(0, QCH)]

    pltpu.sync_copy(tlp.at[sid], tl)

    zero_wbuf()
    for q in range(NQ):
        wbuf_to(acc, q)
    plsc.subcore_barrier()

    def fill_ones(r, x):
        for cc in range(4):
            gb0[r, pl.ds(cc * 16, 16)] = jnp.ones((16,), jnp.float32)
        return x
    lax.fori_loop(0, ECH, fill_ones, None)

    def deg_batch(b, x):
        gb = core * NBH + b
        pltpu.sync_copy(sdp.at[sid, pl.ds(gb * BF, BF)], sdbuf.at[0])
        for k in range(BF):
            pltpu.sync_copy(gb0, acc.at[sdbuf.at[0].at[k].at[1]], add=True)
        return x
    lax.fori_loop(0, NBH, deg_batch, None)
    plsc.subcore_barrier()
    export_partial(1)
    xbarrier(jnp.int32(1))

    for q in range(NQ):
        to_wbuf(acc, q)
        pltpu.sync_copy(
            px_hbm.at[1 - core, 1].at[pl.ds(base + q * QCH, QCH)], gb_hi())

        def dinv_row(r, x):
            for cc in range(4):
                sl = pl.ds(cc * 16, 16)
                wbuf[r, sl] = 1.0 / jnp.maximum(wbuf[r, sl] + gb1[r, sl],
                                                1.0)
            return x
        lax.fori_loop(0, QCH, dinv_row, None)
        pltpu.sync_copy(wbuf, dinv_hbm.at[pl.ds(base + q * QCH, QCH)])
        zero_wbuf()
        wbuf_to(acc, q)

    for q in range(NQ):
        pltpu.sync_copy(p_hbm.at[pl.ds(base + q * QCH, QCH)], gb_lo())

        def neg_row(r, x):
            for cc in range(4):
                sl = pl.ds(cc * 16, 16)
                wbuf[r, sl] = -gb0[r, sl]
            return x
        lax.fori_loop(0, QCH, neg_row, None)
        wbuf_to(cur_h, q)
    plsc.subcore_barrier()

    pltpu.sync_copy(cur_h.at[tl.at[0]], h0t)

    def h0t_row(k, x):
        lchunk = tl[1, pl.ds((k // 16) * 16, 16)]
        lval = lax.gather(
            lchunk, jnp.full((16, 1), k % 16, jnp.int32),
            dimension_numbers=lax.GatherDimensionNumbers(
                offset_dims=(), collapsed_slice_dims=(0,),
                start_index_map=(0,)),
            slice_sizes=(1,),
            mode=lax.GatherScatterMode.PROMISE_IN_BOUNDS)
        for cc in range(4):
            io = lax.iota(jnp.int32, 16) + (cc * 16)
            oh = jnp.where(io == lval, 1.0, 0.0)
            h0t[k, pl.ds(cc * 16, 16)] = oh + h0t[k, pl.ds(cc * 16, 16)]
        return x
    lax.fori_loop(0, TPT, h0t_row, None)
    plsc.subcore_barrier()

    pltpu.sync_copy(h0t, cur_h.at[tl.at[0]])
    plsc.subcore_barrier()

    def idx_ref(sb, k, row):
        return sdbuf.at[sb].at[k].at[row]

    def start_gather(sb, k, s):
        pltpu.async_copy(cur_h.at[idx_ref(sb, k, 0)], gbuf.at[s],
                         semg.at[s])

    def wait_gather(sb, k, s):
        pltpu.make_async_copy(cur_h.at[idx_ref(sb, k, 0)], gbuf.at[s],
                              semg.at[s]).wait()

    def start_scat(sb, k, s):
        pltpu.async_copy(gbuf.at[s], acc.at[idx_ref(sb, k, 1)],
                         sems.at[s], add=True)

    def wait_scat(sb, k, s):
        pltpu.make_async_copy(gbuf.at[s], acc.at[idx_ref(sb, k, 1)],
                              sems.at[s]).wait()

    def stage_idx_sync(b, sb):
        pltpu.sync_copy(sdp.at[sid, pl.ds(b * BF, BF)], sdbuf.at[sb])

    def stage_idx_start(b, sb):
        pltpu.async_copy(sdp.at[sid, pl.ds(b * BF, BF)], sdbuf.at[sb],
                         semi.at[sb])

    def stage_idx_wait(b, sb):
        pltpu.make_async_copy(sdp.at[sid, pl.ds(b * BF, BF)],
                              sdbuf.at[sb], semi.at[sb]).wait()

    def scatter_phase():
        b0 = core * NBH
        stage_idx_sync(b0, 0)
        stage_idx_start(b0 + 1, 1)

        def batch(b, x):
            sb = lax.rem(b - b0, 2)
            nsb = lax.rem(b - b0 + 1, 2)
            for k in range(BF):
                s = k % NSLOT
                if k >= NSLOT:
                    wait_scat(sb, k - NSLOT, s)
                start_gather(sb, k, s)
                if k >= 1:
                    wait_gather(sb, k - 1, (k - 1) % NSLOT)
                    start_scat(sb, k - 1, (k - 1) % NSLOT)
            wait_gather(sb, BF - 1, (BF - 1) % NSLOT)
            start_scat(sb, BF - 1, (BF - 1) % NSLOT)
            for t in range(BF - NSLOT, BF):
                wait_scat(sb, t, t % NSLOT)

            @pl.when(b + 2 < b0 + NBH)
            def _():
                stage_idx_start(b + 2, sb)

            @pl.when(b + 1 < b0 + NBH)
            def _():
                stage_idx_wait(b + 1, nsb)
            return x
        lax.fori_loop(b0, b0 + NBH, batch, None)

    gb2 = gbuf.at[2]

    def scale_chunk(q, slot, second):
        to_wbuf(acc, q)
        pltpu.sync_copy(dinv_hbm.at[pl.ds(base + q * QCH, QCH)], gb_lo())
        pltpu.sync_copy(
            px_hbm.at[1 - core, slot].at[pl.ds(base + q * QCH, QCH)],
            gb_hi())
        if second:
            pltpu.sync_copy(h0b_hbm.at[pl.ds(base + q * QCH, QCH)],
                            gb2.at[pl.ds(0, QCH)])

        def srow(r, x):
            for cc in range(4):
                sl = pl.ds(cc * 16, 16)
                tot = (wbuf[r, sl] + gb1[r, sl]) * gb0[r, sl]
                if second:
                    wbuf[r, sl] = tot * 0.9 + gb2[r, sl] * 0.1
                else:
                    wbuf[r, sl] = tot
            return x
        lax.fori_loop(0, QCH, srow, None)
        wbuf_to(cur_h, q)
        zero_wbuf()
        wbuf_to(acc, q)

    def loop1_step(s, carry):
        t = 2 + s
        slot = lax.rem(t, 2)
        scatter_phase()
        plsc.subcore_barrier()
        export_partial(slot)
        xbarrier(t)
        for q in range(NQ):
            scale_chunk(q, slot, False)
        plsc.subcore_barrier()
        pltpu.sync_copy(h0t, cur_h.at[tl.at[0]])
        plsc.subcore_barrier()
        return carry
    lax.fori_loop(0, DEPTH, loop1_step, None)

    for q in range(NQ):
        to_wbuf(cur_h, q)
        pltpu.sync_copy(p_hbm.at[pl.ds(base + q * QCH, QCH)], gb_lo())

        def arow(r, x):
            for cc in range(4):
                sl = pl.ds(cc * 16, 16)
                wbuf[r, sl] = wbuf[r, sl] + gb0[r, sl]
            return x
        lax.fori_loop(0, QCH, arow, None)
        wbuf_to(cur_h, q)
        pltpu.sync_copy(wbuf, h0b_hbm.at[pl.ds(base + q * QCH, QCH)])
    plsc.subcore_barrier()

    def loop2_step(s, carry):
        t = 12 + s
        slot = lax.rem(t, 2)
        scatter_phase()
        plsc.subcore_barrier()
        export_partial(slot)
        xbarrier(t)
        for q in range(NQ):
            scale_chunk(q, slot, True)
        plsc.subcore_barrier()
        return carry
    lax.fori_loop(0, DEPTH, loop2_step, None)


def _sc_diffusion(p_pad, sdp, tlp):
    mesh = plsc.VectorSubcoreMesh(core_axis_name="c", subcore_axis_name="s")
    kfn = pl.kernel(
        _sc_body,
        out_type=(jax.ShapeDtypeStruct((2, NPAD, C), jnp.float32),
                  jax.ShapeDtypeStruct((2, NPAD, C), jnp.float32),
                  jax.ShapeDtypeStruct((NPAD, C), jnp.float32),
                  jax.ShapeDtypeStruct((2, 2, NPAD, C), jnp.float32),
                  jax.ShapeDtypeStruct((2, 1, 64), jnp.int32)),
        mesh=mesh,
        compiler_params=pltpu.CompilerParams(use_tc_tiling_on_sc=False),
        scratch_types=[
            pltpu.VMEM_SHARED((NPAD, C), jnp.float32),
            pltpu.VMEM((NSLOT, ECH, C), jnp.float32),
            pltpu.VMEM((QCH, C), jnp.float32),
            pltpu.VMEM((TPT, C), jnp.float32),
            pltpu.VMEM((2, BF, 2, ECH), jnp.int32),
            pltpu.VMEM((2, TPT), jnp.int32),
            pltpu.VMEM((1, 64), jnp.int32),
            pltpu.VMEM((1, 64), jnp.int32),
            pltpu.SemaphoreType.DMA((NSLOT,)),
            pltpu.SemaphoreType.DMA((NSLOT,)),
            pltpu.SemaphoreType.DMA((2,)),
            pltpu.SemaphoreType.REGULAR,
        ],
    )
    return kfn(p_pad, sdp, tlp)


def kernel(x, edge_index, train_idx, labels, W1, b1, W2, b2):
    p = _mlp_softmax(x, W1, b1, W2, b2)
    p_pad = jnp.pad(p, ((0, NPAD - N), (0, 0)))

    src = jnp.pad(edge_index[0].astype(jnp.int32), (0, EPAD - E))
    dst = jnp.pad(edge_index[1].astype(jnp.int32), (0, EPAD - E),
                  constant_values=PAD_ROW)
    sdp = jnp.stack([src.reshape(NSUB, NCHUNK, ECH),
                     dst.reshape(NSUB, NCHUNK, ECH)], axis=2)
    tlp = jnp.stack(
        [jnp.pad(train_idx.astype(jnp.int32), (0, NSUB * TPT - NTRAIN),
                 constant_values=PAD_ROW).reshape(NSUB, TPT),
         jnp.pad(labels.astype(jnp.int32),
                 (0, NSUB * TPT - NTRAIN)).reshape(NSUB, TPT)], axis=1)

    cur2, _, _, _, _ = _sc_diffusion(p_pad, sdp, tlp)
    out_pre = jnp.concatenate([cur2[0, :NPAD // 2], cur2[1, NPAD // 2:N]])
    return _log1p(out_pre)
